# Initial kernel scaffold; baseline (speedup 1.0000x reference)
import jax
import jax.numpy as jnp
from jax.experimental import pallas as pl


def kernel(x, edge_index, rel_matrix, train_model, W_gat, a_src, a_dst, b_gat, W_gcn1, b_gcn1, W_gcn2, b_gcn2, W_mlp1, b_mlp1, W_mlp2, b_mlp2, W_mlp3):
    raise NotImplementedError("write your pallas kernel here")


# speedup vs baseline: 13.0359x; 13.0359x over previous
import functools

import jax
import jax.numpy as jnp
from jax import lax
from jax.experimental import pallas as pl
from jax.experimental.pallas import tpu as pltpu
from jax.experimental.pallas import tpu_sc as plsc

N = 10000
F_IN = 128
HEADS = 8
C = 64
HC = HEADS * C
R = 500
D = 400
Nt = 10240
E_RAW = 320000
E_TOT = E_RAW + N
E_PAD = 331776
NC, NS, LN = 2, 16, 16
EB = 128
BLK_ALL = E_PAD // NS // EB
BLK_HALF = BLK_ALL // 2
RPT = Nt // NS

_f32 = jnp.float32
_i32 = jnp.int32


_SC_MESH = plsc.VectorSubcoreMesh(core_axis_name="c", subcore_axis_name="s")


_GDN = lax.GatherDimensionNumbers(offset_dims=(), collapsed_slice_dims=(0,),
                                  start_index_map=(0,))


def _take16(vec, idx):
    return lax.gather(vec, idx[:, None], _GDN, slice_sizes=(1,),
                      mode=lax.GatherScatterMode.PROMISE_IN_BOUNDS)


@functools.partial(
    pl.kernel,
    out_type=(jax.ShapeDtypeStruct((E_PAD, 16), _f32),
              jax.ShapeDtypeStruct((Nt, 16), _f32)),
    mesh=_SC_MESH,
    compiler_params=pltpu.CompilerParams(use_tc_tiling_on_sc=False),
    scratch_types=[
        pltpu.VMEM((EB,), _i32),
        pltpu.VMEM((EB,), _i32),
        pltpu.VMEM((EB, 16), _f32),
        pltpu.VMEM((EB, 16), _f32),
        pltpu.VMEM((EB, 16), _f32),
        pltpu.VMEM_SHARED((Nt, 16), _f32),
        pltpu.SemaphoreType.DMA,
        pltpu.SemaphoreType.DMA,
    ],
)
def _sc_edge_stats(als_hbm, ald_hbm, src_hbm, dst_hbm, z16_hbm, ex_out, den_out,
                   sidx, didx, srow, drow, exb, den_sh, sem1, sem2):
    cid = lax.axis_index("c")
    sid = lax.axis_index("s")

    @pl.when(sid == 0)
    def _():
        pltpu.sync_copy(z16_hbm, den_sh)
    plsc.subcore_barrier()

    write_ex = jnp.logical_or(jnp.logical_and(cid == 0, sid < 8),
                              jnp.logical_and(cid == 1, sid >= 8))
    tile_base = sid * (E_PAD // NS)

    def body(b, carry):
        base = tile_base + b * EB
        pltpu.sync_copy(src_hbm.at[pl.ds(base, EB)], sidx)
        pltpu.sync_copy(dst_hbm.at[pl.ds(base, EB)], didx)
        cp1 = pltpu.async_copy(als_hbm.at[sidx], srow, sem1)
        cp2 = pltpu.async_copy(ald_hbm.at[didx], drow, sem2)
        cp1.wait()
        cp2.wait()

        def inner(k, c2):
            e = srow[k] + drow[k]
            e = jnp.where(e >= 0.0, e, 0.2 * e)
            exb[k] = jnp.exp(e)
            return c2
        lax.fori_loop(0, EB, inner, 0, unroll=4)

        pltpu.sync_copy(exb, den_sh.at[didx], add=True)

        @pl.when(write_ex)
        def _():
            pltpu.sync_copy(exb, ex_out.at[pl.ds(base, EB)])
        return carry
    lax.fori_loop(0, BLK_ALL, body, 0)

    plsc.subcore_barrier()
    row0 = cid * (Nt // 2) + sid * (Nt // 2 // NS)
    pltpu.sync_copy(den_sh.at[pl.ds(row0, Nt // 2 // NS)],
                    den_out.at[pl.ds(row0, Nt // 2 // NS)])


@functools.partial(
    pl.kernel,
    out_type=jax.ShapeDtypeStruct((4, Nt, 128), _f32),
    mesh=_SC_MESH,
    compiler_params=pltpu.CompilerParams(use_tc_tiling_on_sc=False),
    scratch_types=[
        pltpu.VMEM((EB,), _i32),
        pltpu.VMEM((EB,), _i32),
        pltpu.VMEM((EB, 128), _f32),
        pltpu.VMEM((EB, 16), _f32),
        pltpu.VMEM((EB, 16), _f32),
        pltpu.VMEM_SHARED((Nt, 128), _f32),
        pltpu.SemaphoreType.DMA,
        pltpu.SemaphoreType.DMA,
    ],
)
def _sc_gat_agg(h4_hbm, ex_hbm, den_hbm, src_hbm, dst_hbm, z128_hbm, out_hbm,
                sidx, didx, rows, exb, denb, out_sh, sem1, sem2):
    cid = lax.axis_index("c")
    sid = lax.axis_index("s")
    tile_base = sid * (E_PAD // NS)

    for fc in range(4):
        @pl.when(cid == fc % 2)
        def _(fc=fc):
            @pl.when(sid == 0)
            def _():
                pltpu.sync_copy(z128_hbm, out_sh)
            plsc.subcore_barrier()
            tbl = h4_hbm.at[fc]
            idx0 = jnp.full((LN,), 2 * fc, _i32)
            idx1 = jnp.full((LN,), 2 * fc + 1, _i32)

            def body(b, carry):
                base = tile_base + b * EB
                pltpu.sync_copy(src_hbm.at[pl.ds(base, EB)], sidx)
                pltpu.sync_copy(dst_hbm.at[pl.ds(base, EB)], didx)
                cp1 = pltpu.async_copy(tbl.at[sidx], rows, sem1)
                cp2 = pltpu.async_copy(den_hbm.at[didx], denb, sem2)
                pltpu.sync_copy(ex_hbm.at[pl.ds(base, EB)], exb)
                cp1.wait()
                cp2.wait()

                def inner(k, c2):
                    coef = exb[k] / (denb[k] + 1e-16)
                    c0 = _take16(coef, idx0)
                    c1 = _take16(coef, idx1)
                    for j in range(4):
                        rows[k, pl.ds(16 * j, 16)] = rows[k, pl.ds(16 * j, 16)] * c0
                    for j in range(4, 8):
                        rows[k, pl.ds(16 * j, 16)] = rows[k, pl.ds(16 * j, 16)] * c1
                    return c2
                lax.fori_loop(0, EB, inner, 0, unroll=2)

                pltpu.sync_copy(rows, out_sh.at[didx], add=True)
                return carry
            lax.fori_loop(0, BLK_ALL, body, 0)

            plsc.subcore_barrier()
            row0 = sid * RPT
            pltpu.sync_copy(out_sh.at[pl.ds(row0, RPT)],
                            out_hbm.at[fc].at[pl.ds(row0, RPT)])
            plsc.subcore_barrier()


def _make_sc_gcn_agg(nch, split_edges):
    n_out = 2 if split_edges else nch

    @functools.partial(
        pl.kernel,
        out_type=jax.ShapeDtypeStruct((n_out, Nt, 128), _f32),
        mesh=_SC_MESH,
        compiler_params=pltpu.CompilerParams(use_tc_tiling_on_sc=False),
        scratch_types=[
            pltpu.VMEM((EB,), _i32),
            pltpu.VMEM((EB,), _i32),
            pltpu.VMEM((EB, 128), _f32),
            pltpu.VMEM((EB, 16), _f32),
            pltpu.VMEM((EB, 16), _f32),
            pltpu.VMEM_SHARED((Nt, 128), _f32),
            pltpu.SemaphoreType.DMA,
            pltpu.SemaphoreType.DMA,
        ],
    )
    def k(xw_hbm, dis_hbm, src_hbm, dst_hbm, z128_hbm, out_hbm,
          sidx, didx, rows, dsb, ddb, out_sh, sem1, sem2):
        cid = lax.axis_index("c")
        sid = lax.axis_index("s")

        def run_chunk(fc, oc, tile_base, nblk):
            @pl.when(sid == 0)
            def _():
                pltpu.sync_copy(z128_hbm, out_sh)
            plsc.subcore_barrier()
            tbl = xw_hbm.at[fc]

            def body(b, carry):
                base = tile_base + b * EB
                pltpu.sync_copy(src_hbm.at[pl.ds(base, EB)], sidx)
                pltpu.sync_copy(dst_hbm.at[pl.ds(base, EB)], didx)
                cp1 = pltpu.async_copy(tbl.at[sidx], rows, sem1)
                cp2 = pltpu.async_copy(dis_hbm.at[sidx], dsb, sem2)
                cp1.wait()
                cp2.wait()
                cp3 = pltpu.async_copy(dis_hbm.at[didx], ddb, sem2)
                cp3.wait()

                def inner(k2, c2):
                    nv = dsb[k2] * ddb[k2]
                    for j in range(8):
                        rows[k2, pl.ds(16 * j, 16)] = rows[k2, pl.ds(16 * j, 16)] * nv
                    return c2
                lax.fori_loop(0, EB, inner, 0, unroll=2)

                pltpu.sync_copy(rows, out_sh.at[didx], add=True)
                return carry
            lax.fori_loop(0, nblk, body, 0)

            plsc.subcore_barrier()
            row0 = sid * RPT
            pltpu.sync_copy(out_sh.at[pl.ds(row0, RPT)],
                            out_hbm.at[oc].at[pl.ds(row0, RPT)])
            plsc.subcore_barrier()

        if split_edges:
            for c in range(2):
                @pl.when(cid == c)
                def _(c=c):
                    run_chunk(0, c, c * (E_PAD // 2) + sid * (E_PAD // 2 // NS),
                              BLK_HALF)
        else:
            for fc in range(nch):
                @pl.when(cid == fc % 2)
                def _(fc=fc):
                    run_chunk(fc, fc, sid * (E_PAD // NS), BLK_ALL)
    return k


_sc_gcn1_agg = _make_sc_gcn_agg(2, False)
_sc_gcn2_agg = _make_sc_gcn_agg(1, True)


_GRID_R = 8
_BR = Nt // _GRID_R


def _tc1_body(x_ref, w_ref, a2_ref, h4_ref, als_ref, ald_ref):
    h = jnp.dot(x_ref[...], w_ref[...], preferred_element_type=_f32)
    al2 = jnp.dot(h, a2_ref[...], preferred_element_type=_f32)
    als_ref[...] = al2[:, :16]
    ald_ref[...] = al2[:, 16:]
    for c in range(4):
        h4_ref[c] = h[:, 128 * c:128 * (c + 1)]


def _tc1(xp, W_gat, A2):
    return pl.pallas_call(
        _tc1_body,
        grid=(_GRID_R,),
        in_specs=[
            pl.BlockSpec((_BR, F_IN), lambda r: (r, 0)),
            pl.BlockSpec((F_IN, HC), lambda r: (0, 0)),
            pl.BlockSpec((HC, 32), lambda r: (0, 0)),
        ],
        out_specs=[
            pl.BlockSpec((4, _BR, 128), lambda r: (0, r, 0)),
            pl.BlockSpec((_BR, 16), lambda r: (r, 0)),
            pl.BlockSpec((_BR, 16), lambda r: (r, 0)),
        ],
        out_shape=[
            jax.ShapeDtypeStruct((4, Nt, 128), _f32),
            jax.ShapeDtypeStruct((Nt, 16), _f32),
            jax.ShapeDtypeStruct((Nt, 16), _f32),
        ],
    )(xp, W_gat, A2)


def _elu(v):
    return jnp.where(v > 0.0, v, jnp.exp(jnp.minimum(v, 0.0)) - 1.0)


def _tc2_body(p4_ref, b_ref, w_ref, den_ref, xw2_ref, dis_ref):
    g = jnp.concatenate([p4_ref[c] for c in range(4)], axis=-1)
    h1 = _elu(g + b_ref[...])
    xw = jnp.dot(h1, w_ref[...], preferred_element_type=_f32)
    for c in range(2):
        xw2_ref[c] = xw[:, 128 * c:128 * (c + 1)]
    deg = den_ref[:, 8:9]
    dis = jnp.where(deg > 0.0, lax.rsqrt(jnp.maximum(deg, 1e-12)), 0.0)
    dis_ref[...] = jnp.broadcast_to(dis, dis_ref.shape)


def _tc2(p4, b_gat, W_gcn1, den):
    return pl.pallas_call(
        _tc2_body,
        grid=(_GRID_R,),
        in_specs=[
            pl.BlockSpec((4, _BR, 128), lambda r: (0, r, 0)),
            pl.BlockSpec((1, HC), lambda r: (0, 0)),
            pl.BlockSpec((HC, 256), lambda r: (0, 0)),
            pl.BlockSpec((_BR, 16), lambda r: (r, 0)),
        ],
        out_specs=[
            pl.BlockSpec((2, _BR, 128), lambda r: (0, r, 0)),
            pl.BlockSpec((_BR, 16), lambda r: (r, 0)),
        ],
        out_shape=[
            jax.ShapeDtypeStruct((2, Nt, 128), _f32),
            jax.ShapeDtypeStruct((Nt, 16), _f32),
        ],
    )(p4, b_gat, W_gcn1, den)


def _tc3_body(p2_ref, b_ref, w_ref, xw_ref):
    g = jnp.concatenate([p2_ref[c] for c in range(2)], axis=-1)
    h2 = _elu(g + b_ref[...])
    xw_ref[...] = jnp.dot(h2, w_ref[...], preferred_element_type=_f32)


def _tc3(p2, b_gcn1, W_gcn2):
    return pl.pallas_call(
        _tc3_body,
        grid=(_GRID_R,),
        in_specs=[
            pl.BlockSpec((2, _BR, 128), lambda r: (0, r, 0)),
            pl.BlockSpec((1, 256), lambda r: (0, 0)),
            pl.BlockSpec((256, 128), lambda r: (0, 0)),
        ],
        out_specs=pl.BlockSpec((_BR, 128), lambda r: (r, 0)),
        out_shape=jax.ShapeDtypeStruct((Nt, 128), _f32),
    )(p2, b_gcn1, W_gcn2)


def _tc4_body(p2_ref, b_ref, w1t_ref, w1b_ref, pq_ref):
    h3 = _elu(p2_ref[0] + p2_ref[1] + b_ref[...])
    o = jnp.maximum(h3, 0.0)
    p = jnp.dot(o, w1t_ref[...], preferred_element_type=_f32)
    q = jnp.dot(o, w1b_ref[...], preferred_element_type=_f32)
    pq_ref[...] = jnp.concatenate([p, q], axis=-1)


def _tc4(p2, b_gcn2, W1t, W1b):
    return pl.pallas_call(
        _tc4_body,
        grid=(_GRID_R,),
        in_specs=[
            pl.BlockSpec((2, _BR, 128), lambda r: (0, r, 0)),
            pl.BlockSpec((1, 128), lambda r: (0, 0)),
            pl.BlockSpec((128, 64), lambda r: (0, 0)),
            pl.BlockSpec((128, 64), lambda r: (0, 0)),
        ],
        out_specs=pl.BlockSpec((_BR, 128), lambda r: (r, 0)),
        out_shape=jax.ShapeDtypeStruct((Nt, 128), _f32),
    )(p2, b_gcn2, W1t, W1b)


_RP = 512
_BI = 64


def _tc5_body(a_ref, b_ref, b1_ref, w2_ref, b2_ref, w3_ref, out_ref):
    s = a_ref[...][:, None, :] + b_ref[...][None, :, :] + b1_ref[...][None, :, :]
    m1 = jnp.where(s > 0.0, s, 0.01 * s).reshape(_BI * D, 64)
    m2 = jnp.dot(m1, w2_ref[...], preferred_element_type=_f32) + b2_ref[...]
    m2 = jnp.where(m2 > 0.0, m2, 0.01 * m2)
    z = jnp.dot(m2, w3_ref[...], preferred_element_type=_f32)
    out_ref[...] = jax.nn.sigmoid(z)


def _tc5(A, B, b1, W2, b2, W3):
    return pl.pallas_call(
        _tc5_body,
        grid=(_RP // _BI,),
        in_specs=[
            pl.BlockSpec((_BI, 64), lambda i: (i, 0)),
            pl.BlockSpec((D, 64), lambda i: (0, 0)),
            pl.BlockSpec((1, 64), lambda i: (0, 0)),
            pl.BlockSpec((64, 128), lambda i: (0, 0)),
            pl.BlockSpec((1, 128), lambda i: (0, 0)),
            pl.BlockSpec((128, 1), lambda i: (0, 0)),
        ],
        out_specs=pl.BlockSpec((_BI * D, 1), lambda i: (i, 0)),
        out_shape=jax.ShapeDtypeStruct((_RP * D, 1), _f32),
    )(A, B, b1, W2, b2, W3)


def kernel(x, edge_index, rel_matrix, train_model, W_gat, a_src, a_dst, b_gat,
           W_gcn1, b_gcn1, W_gcn2, b_gcn2, W_mlp1, b_mlp1, W_mlp2, b_mlp2,
           W_mlp3):
    loop = jnp.arange(N, dtype=_i32)
    padv = jnp.full((E_PAD - E_TOT,), N, dtype=_i32)
    src = jnp.concatenate([edge_index[0].astype(_i32), loop, padv])
    dst = jnp.concatenate([edge_index[1].astype(_i32), loop, padv])
    xp = jnp.zeros((Nt, F_IN), _f32).at[:N].set(x)

    rows512 = jnp.arange(HC)
    head = jnp.repeat(jnp.arange(HEADS), C)
    A_s = jnp.zeros((HC, 16), _f32).at[rows512, head].set(a_src.reshape(-1))
    A_d = jnp.zeros((HC, 16), _f32).at[rows512, head].set(a_dst.reshape(-1))
    A2 = jnp.concatenate([A_s, A_d], axis=1)

    z16 = jnp.zeros((Nt, 16), _f32)
    z128 = jnp.zeros((Nt, 128), _f32)

    h4, als, ald = _tc1(xp, W_gat, A2)
    ex, den = _sc_edge_stats(als, ald, src, dst, z16)
    gat4 = _sc_gat_agg(h4, ex, den, src, dst, z128)
    xw2ch, dis = _tc2(gat4, b_gat.reshape(1, HC), W_gcn1, den)

    g1 = _sc_gcn1_agg(xw2ch, dis, src, dst, z128)
    xw2 = _tc3(g1, b_gcn1.reshape(1, 256), W_gcn2)

    g2 = _sc_gcn2_agg(xw2.reshape(1, Nt, 128), dis, src, dst, z128)
    pq = _tc4(g2, b_gcn2.reshape(1, 128), W_mlp1[:128], W_mlp1[128:])

    A = pq[:_RP, :64]
    B = pq[R:R + D, 64:]
    pred_pad = _tc5(A, B, b_mlp1.reshape(1, 64), W_mlp2,
                    b_mlp2.reshape(1, 128), W_mlp3)
    pred = pred_pad[:R * D]
    labels = rel_matrix.reshape(-1, 1)
    return (pred, labels)
